# baseline (speedup 1.0000x reference)
import functools

import jax
import jax.numpy as jnp
from jax import lax
from jax.experimental import pallas as pl
from jax.experimental.pallas import tpu as pltpu
from jax.experimental.pallas import tpu_sc as plsc

N_NODES = 10000
N_EDGES = 320000
D = 128

NW = 32
NLOC = 320
NPAD = NW * NLOC
TRASH = NLOC
ACCR = NLOC + 1
LANES = 16
SUBCAP = 1024
CAP = LANES * SUBCAP
CHUNK = 256
NBUF = 4
LANES2 = 32
RPT2 = D // LANES2
DW = D // 2
ECHUNK = 16000
RPT = D // LANES


def _edge_partition_kernel():
    mesh = plsc.VectorSubcoreMesh(core_axis_name="c", subcore_axis_name="s")

    @functools.partial(
        pl.kernel,
        out_type=(
            jax.ShapeDtypeStruct((NW, CAP), jnp.int32),
            jax.ShapeDtypeStruct((NW, 128), jnp.int32),
        ),
        mesh=mesh,
        compiler_params=pltpu.CompilerParams(needs_layout_passes=False),
        scratch_types=[
            pltpu.VMEM((CAP,), jnp.int32),
            pltpu.VMEM((CAP,), jnp.int32),
            pltpu.VMEM((ECHUNK,), jnp.int32),
            pltpu.VMEM((ECHUNK,), jnp.int32),
            pltpu.VMEM((128,), jnp.int32),
        ],
    )
    def part(src_hbm, dst_hbm, lists_hbm, counts_hbm, lv, lvc, srcb, dstb,
             cv):
        wid = lax.axis_index("s") * 2 + lax.axis_index("c")
        lo = wid * NLOC
        pad = jnp.full((LANES,), TRASH, jnp.int32)

        def fill(i, _):
            lv[pl.ds(i * LANES, LANES)] = pad
            return 0

        lax.fori_loop(0, SUBCAP, fill, 0)

        lanebase = lax.iota(jnp.int32, LANES) * SUBCAP

        def outer(k, cnt):
            pltpu.sync_copy(src_hbm.at[pl.ds(k * ECHUNK, ECHUNK)], srcb)
            pltpu.sync_copy(dst_hbm.at[pl.ds(k * ECHUNK, ECHUNK)], dstb)

            def inner(j, cnt):
                for u in range(2):
                    sv = srcb[pl.ds(j * 2 * LANES + u * LANES, LANES)]
                    dv = dstb[pl.ds(j * 2 * LANES + u * LANES, LANES)]
                    dl = dv - lo
                    m = (dl >= 0) & (dl < NLOC)
                    pk = (sv << 9) | jnp.where(m, dl, 0)
                    plsc.store_scatter(lv, [lanebase + cnt], pk, mask=m)
                    cnt = cnt + jnp.where(m, 1, 0).astype(jnp.int32)
                return cnt

            return lax.fori_loop(0, ECHUNK // (2 * LANES), inner, cnt)

        cnt = lax.fori_loop(0, N_EDGES // ECHUNK, outer,
                            jnp.zeros((LANES,), jnp.int32))

        base = jnp.int32(0)
        for l in range(LANES):
            nv = (cnt[l] + (LANES - 1)) >> 4
            src_base = l * SUBCAP

            def cp(j, _, b=base, s=src_base):
                lvc[pl.ds(b + j * LANES, LANES)] = lv[pl.ds(s + j * LANES,
                                                            LANES)]
                return 0

            lax.fori_loop(0, nv, cp, 0)
            base = base + nv * LANES
        for j in range(CHUNK // LANES):
            lvc[pl.ds(base + j * LANES, LANES)] = pad

        cv[pl.ds(0, LANES)] = jnp.full((LANES,), 1, jnp.int32) * base
        pltpu.sync_copy(lvc, lists_hbm.at[wid])
        pltpu.sync_copy(cv, counts_hbm.at[wid])

    return part


def _max_agg_kernel():
    mesh = plsc.VectorSubcoreMesh(core_axis_name="c", subcore_axis_name="s")

    @functools.partial(
        pl.kernel,
        out_type=jax.ShapeDtypeStruct((NPAD * DW,), jnp.int32),
        mesh=mesh,
        compiler_params=pltpu.CompilerParams(needs_layout_passes=False,
                                             use_tc_tiling_on_sc=False),
        scratch_types=[
            pltpu.VMEM((CAP,), jnp.int32),
            pltpu.VMEM((128,), jnp.int32),
            pltpu.VMEM((ACCR * DW,), jnp.int32),
        ] + [
            t for _ in range(NBUF) for t in (
                pltpu.VMEM((CHUNK,), jnp.int32),
                pltpu.VMEM((CHUNK,), jnp.int32),
                pltpu.VMEM((CHUNK, DW), jnp.int32),
                pltpu.SemaphoreType.DMA,
            )
        ],
    )
    def agg(h_hbm, lists_hbm, counts_hbm, out_hbm, lv, cv, acc, *bufargs):
        wid = lax.axis_index("s") * 2 + lax.axis_index("c")
        pltpu.sync_copy(lists_hbm.at[wid], lv)
        pltpu.sync_copy(counts_hbm.at[wid], cv)
        total = cv[pl.ds(0, LANES)][0]
        nch = (total + (CHUNK - 1)) // CHUNK

        ninf = jnp.full((LANES2,), -jnp.inf, jnp.bfloat16)
        zero = jnp.zeros((LANES2,), jnp.bfloat16)
        ninf32 = plsc.bitcast(ninf, jnp.int32)

        def initrow(rr, _):
            for k in range(RPT2):
                acc[pl.ds(rr * DW + k * LANES, LANES)] = ninf32
            return 0

        lax.fori_loop(0, ACCR, initrow, 0)

        bufs = tuple(tuple(bufargs[4 * b:4 * b + 4]) for b in range(NBUF))

        def issue(b, c):
            sidx, dloc, msg, sem = bufs[b]
            base = c * CHUNK

            def unpack(j, _):
                pk = lv[pl.ds(base + j * LANES, LANES)]
                sidx[pl.ds(j * LANES, LANES)] = pk >> 9
                dloc[pl.ds(j * LANES, LANES)] = pk & (2 ** 9 - 1)
                return 0

            lax.fori_loop(0, CHUNK // LANES, unpack, 0)
            pltpu.async_copy(h_hbm.at[sidx], msg, sem)

        for b in range(NBUF):
            @pl.when(nch > b)
            def _(b=b):
                issue(b, b)

        def louter(c2, _):
            for b in range(NBUF):
                c = c2 * NBUF + b
                sidx, dloc, msg, sem = bufs[b]

                @pl.when(c < nch)
                def _(c=c, sidx=sidx, dloc=dloc, msg=msg, sem=sem, b=b):
                    pltpu.make_async_copy(h_hbm.at[sidx], msg, sem).wait()

                    def do_group(g, _):
                        dv = dloc[pl.ds(g * LANES, LANES)]
                        for lane in range(LANES):
                            off = dv[lane] * DW
                            erow = g * LANES + lane
                            for r in range(RPT2):
                                a = plsc.bitcast(
                                    acc[pl.ds(off + r * LANES, LANES)],
                                    jnp.bfloat16)
                                mv = plsc.bitcast(
                                    msg[erow, pl.ds(r * LANES, LANES)],
                                    jnp.bfloat16)
                                acc[pl.ds(off + r * LANES, LANES)] = (
                                    plsc.bitcast(jnp.maximum(a, mv),
                                                 jnp.int32))
                        return 0

                    lax.fori_loop(0, CHUNK // LANES, do_group, 0)

                    @pl.when(c + NBUF < nch)
                    def _():
                        issue(b, c + NBUF)

            return 0

        lax.fori_loop(0, (nch + NBUF - 1) // NBUF, louter, 0)

        def fixrow(rr, _):
            for k in range(RPT2):
                v = plsc.bitcast(acc[pl.ds(rr * DW + k * LANES, LANES)],
                                 jnp.bfloat16)
                acc[pl.ds(rr * DW + k * LANES, LANES)] = (
                    plsc.bitcast(jnp.where(v == ninf, zero, v), jnp.int32))
            return 0

        lax.fori_loop(0, NLOC, fixrow, 0)
        pltpu.sync_copy(acc.at[pl.ds(0, NLOC * DW)],
                        out_hbm.at[pl.ds(wid * NLOC * DW, NLOC * DW)])

    return agg


_part = _edge_partition_kernel()
_agg = _max_agg_kernel()

_ROWS = NPAD // 4


def _tc_linear(h, w, b, silu_in):
    def body(h_ref, w_ref, b_ref, o_ref):
        hv = h_ref[...].astype(jnp.float32)
        if silu_in:
            hv = hv / (1.0 + jnp.exp(-hv))
        o_ref[...] = (lax.dot_general(
            hv, w_ref[...], (((1,), (1,)), ((), ())),
            preferred_element_type=jnp.float32,
            precision=lax.Precision.HIGHEST,
        ) + b_ref[...]).astype(jnp.bfloat16)

    return pl.pallas_call(
        body,
        grid=(NPAD // _ROWS,),
        in_specs=[
            pl.BlockSpec((_ROWS, D), lambda i: (i, 0)),
            pl.BlockSpec((D, D), lambda i: (0, 0)),
            pl.BlockSpec((1, D), lambda i: (0, 0)),
        ],
        out_specs=pl.BlockSpec((_ROWS, D), lambda i: (i, 0)),
        out_shape=jax.ShapeDtypeStruct((NPAD, D), jnp.bfloat16),
    )(h, w, b.reshape(1, D))


def kernel(x, edge_index, W0, b0, W1, b1, W2, b2, W3, b3, W4, b4):
    src = edge_index[0].astype(jnp.int32)
    dst = edge_index[1].astype(jnp.int32)
    xp = jnp.zeros((NPAD, D), jnp.float32).at[:N_NODES].set(x)

    lists, counts = _part(src, dst)

    ws = [W0, W1, W2, W3, W4]
    bs = [b0, b1, b2, b3, b4]
    h = xp
    for i in range(5):
        g = _tc_linear(h, ws[i], bs[i], silu_in=(i > 0))
        gp = lax.bitcast_convert_type(g.reshape(NPAD, DW, 2), jnp.int32)
        a = _agg(gp, lists, counts)
        h = lax.bitcast_convert_type(a.reshape(NPAD, DW),
                                     jnp.bfloat16).reshape(NPAD, D)
    return h[:N_NODES].astype(jnp.float32)


# speedup vs baseline: 1.1268x; 1.1268x over previous
import functools

import jax
import jax.numpy as jnp
from jax import lax
from jax.experimental import pallas as pl
from jax.experimental.pallas import tpu as pltpu
from jax.experimental.pallas import tpu_sc as plsc

N_NODES = 10000
N_EDGES = 320000
D = 128

NW = 32
NLOC = 320
NPAD = NW * NLOC
TRASH = NLOC
ACCR = NLOC + 1
LANES = 16
SUBCAP = 1024
CAP = LANES * SUBCAP
CHUNK = 128
NBUF = 8
LANES2 = 32
RPT2 = D // LANES2
DW = D // 2
ECHUNK = 16000
RPT = D // LANES


def _edge_partition_kernel():
    mesh = plsc.VectorSubcoreMesh(core_axis_name="c", subcore_axis_name="s")

    @functools.partial(
        pl.kernel,
        out_type=(
            jax.ShapeDtypeStruct((NW, CAP), jnp.int32),
            jax.ShapeDtypeStruct((NW, 128), jnp.int32),
        ),
        mesh=mesh,
        compiler_params=pltpu.CompilerParams(needs_layout_passes=False),
        scratch_types=[
            pltpu.VMEM((CAP,), jnp.int32),
            pltpu.VMEM((CAP,), jnp.int32),
            pltpu.VMEM((ECHUNK,), jnp.int32),
            pltpu.VMEM((ECHUNK,), jnp.int32),
            pltpu.VMEM((128,), jnp.int32),
        ],
    )
    def part(src_hbm, dst_hbm, lists_hbm, counts_hbm, lv, lvc, srcb, dstb,
             cv):
        wid = lax.axis_index("s") * 2 + lax.axis_index("c")
        lo = wid * NLOC
        pad = jnp.full((LANES,), TRASH, jnp.int32)

        def fill(i, _):
            lv[pl.ds(i * LANES, LANES)] = pad
            return 0

        lax.fori_loop(0, SUBCAP, fill, 0)

        lanebase = lax.iota(jnp.int32, LANES) * SUBCAP

        def outer(k, cnt):
            pltpu.sync_copy(src_hbm.at[pl.ds(k * ECHUNK, ECHUNK)], srcb)
            pltpu.sync_copy(dst_hbm.at[pl.ds(k * ECHUNK, ECHUNK)], dstb)

            def inner(j, cnt):
                for u in range(2):
                    sv = srcb[pl.ds(j * 2 * LANES + u * LANES, LANES)]
                    dv = dstb[pl.ds(j * 2 * LANES + u * LANES, LANES)]
                    dl = dv - lo
                    m = (dl >= 0) & (dl < NLOC)
                    pk = (sv << 9) | jnp.where(m, dl, 0)
                    plsc.store_scatter(lv, [lanebase + cnt], pk, mask=m)
                    cnt = cnt + jnp.where(m, 1, 0).astype(jnp.int32)
                return cnt

            return lax.fori_loop(0, ECHUNK // (2 * LANES), inner, cnt)

        cnt = lax.fori_loop(0, N_EDGES // ECHUNK, outer,
                            jnp.zeros((LANES,), jnp.int32))

        base = jnp.int32(0)
        for l in range(LANES):
            nv = (cnt[l] + (LANES - 1)) >> 4
            src_base = l * SUBCAP

            def cp(j, _, b=base, s=src_base):
                lvc[pl.ds(b + j * LANES, LANES)] = lv[pl.ds(s + j * LANES,
                                                            LANES)]
                return 0

            lax.fori_loop(0, nv, cp, 0)
            base = base + nv * LANES
        for j in range(CHUNK // LANES):
            lvc[pl.ds(base + j * LANES, LANES)] = pad

        cv[pl.ds(0, LANES)] = jnp.full((LANES,), 1, jnp.int32) * base
        pltpu.sync_copy(lvc, lists_hbm.at[wid])
        pltpu.sync_copy(cv, counts_hbm.at[wid])

    return part


def _max_agg_kernel():
    mesh = plsc.VectorSubcoreMesh(core_axis_name="c", subcore_axis_name="s")

    @functools.partial(
        pl.kernel,
        out_type=jax.ShapeDtypeStruct((NPAD * DW,), jnp.int32),
        mesh=mesh,
        compiler_params=pltpu.CompilerParams(needs_layout_passes=False,
                                             use_tc_tiling_on_sc=False),
        scratch_types=[
            pltpu.VMEM((CAP,), jnp.int32),
            pltpu.VMEM((128,), jnp.int32),
            pltpu.VMEM((ACCR * DW,), jnp.int32),
        ] + [
            t for _ in range(NBUF) for t in (
                pltpu.VMEM((CHUNK,), jnp.int32),
                pltpu.VMEM((CHUNK,), jnp.int32),
                pltpu.VMEM((CHUNK, DW), jnp.int32),
                pltpu.SemaphoreType.DMA,
            )
        ],
    )
    def agg(h_hbm, lists_hbm, counts_hbm, out_hbm, lv, cv, acc, *bufargs):
        wid = lax.axis_index("s") * 2 + lax.axis_index("c")
        pltpu.sync_copy(lists_hbm.at[wid], lv)
        pltpu.sync_copy(counts_hbm.at[wid], cv)
        total = cv[pl.ds(0, LANES)][0]
        nch = (total + (CHUNK - 1)) // CHUNK

        ninf = jnp.full((LANES2,), -jnp.inf, jnp.bfloat16)
        zero = jnp.zeros((LANES2,), jnp.bfloat16)
        ninf32 = plsc.bitcast(ninf, jnp.int32)

        def initrow(rr, _):
            for k in range(RPT2):
                acc[pl.ds(rr * DW + k * LANES, LANES)] = ninf32
            return 0

        lax.fori_loop(0, ACCR, initrow, 0)

        bufs = tuple(tuple(bufargs[4 * b:4 * b + 4]) for b in range(NBUF))

        def issue(b, c):
            sidx, dloc, msg, sem = bufs[b]
            base = c * CHUNK

            def unpack(j, _):
                pk = lv[pl.ds(base + j * LANES, LANES)]
                sidx[pl.ds(j * LANES, LANES)] = pk >> 9
                dloc[pl.ds(j * LANES, LANES)] = pk & (2 ** 9 - 1)
                return 0

            lax.fori_loop(0, CHUNK // LANES, unpack, 0)
            pltpu.async_copy(h_hbm.at[sidx], msg, sem)

        for b in range(NBUF):
            @pl.when(nch > b)
            def _(b=b):
                issue(b, b)

        def louter(c2, _):
            for b in range(NBUF):
                c = c2 * NBUF + b
                sidx, dloc, msg, sem = bufs[b]

                @pl.when(c < nch)
                def _(c=c, sidx=sidx, dloc=dloc, msg=msg, sem=sem, b=b):
                    pltpu.make_async_copy(h_hbm.at[sidx], msg, sem).wait()

                    def do_group(g, _):
                        dv = dloc[pl.ds(g * LANES, LANES)]
                        for lane in range(LANES):
                            off = dv[lane] * DW
                            erow = g * LANES + lane
                            for r in range(RPT2):
                                a = plsc.bitcast(
                                    acc[pl.ds(off + r * LANES, LANES)],
                                    jnp.bfloat16)
                                mv = plsc.bitcast(
                                    msg[erow, pl.ds(r * LANES, LANES)],
                                    jnp.bfloat16)
                                acc[pl.ds(off + r * LANES, LANES)] = (
                                    plsc.bitcast(jnp.maximum(a, mv),
                                                 jnp.int32))
                        return 0

                    lax.fori_loop(0, CHUNK // LANES, do_group, 0)

                    @pl.when(c + NBUF < nch)
                    def _():
                        issue(b, c + NBUF)

            return 0

        lax.fori_loop(0, (nch + NBUF - 1) // NBUF, louter, 0)

        def fixrow(rr, _):
            for k in range(RPT2):
                v = plsc.bitcast(acc[pl.ds(rr * DW + k * LANES, LANES)],
                                 jnp.bfloat16)
                acc[pl.ds(rr * DW + k * LANES, LANES)] = (
                    plsc.bitcast(jnp.where(v == ninf, zero, v), jnp.int32))
            return 0

        lax.fori_loop(0, NLOC, fixrow, 0)
        pltpu.sync_copy(acc.at[pl.ds(0, NLOC * DW)],
                        out_hbm.at[pl.ds(wid * NLOC * DW, NLOC * DW)])

    return agg


_part = _edge_partition_kernel()
_agg = _max_agg_kernel()

_ROWS = NPAD // 4


def _tc_linear(h, w, b, silu_in):
    def body(h_ref, w_ref, b_ref, o_ref):
        hv = h_ref[...].astype(jnp.float32)
        if silu_in:
            hv = hv / (1.0 + jnp.exp(-hv))
        o_ref[...] = (lax.dot_general(
            hv, w_ref[...], (((1,), (1,)), ((), ())),
            preferred_element_type=jnp.float32,
            precision=lax.Precision.HIGHEST,
        ) + b_ref[...]).astype(jnp.bfloat16)

    return pl.pallas_call(
        body,
        grid=(NPAD // _ROWS,),
        in_specs=[
            pl.BlockSpec((_ROWS, D), lambda i: (i, 0)),
            pl.BlockSpec((D, D), lambda i: (0, 0)),
            pl.BlockSpec((1, D), lambda i: (0, 0)),
        ],
        out_specs=pl.BlockSpec((_ROWS, D), lambda i: (i, 0)),
        out_shape=jax.ShapeDtypeStruct((NPAD, D), jnp.bfloat16),
    )(h, w, b.reshape(1, D))


def kernel(x, edge_index, W0, b0, W1, b1, W2, b2, W3, b3, W4, b4):
    src = edge_index[0].astype(jnp.int32)
    dst = edge_index[1].astype(jnp.int32)
    xp = jnp.zeros((NPAD, D), jnp.float32).at[:N_NODES].set(x)

    lists, counts = _part(src, dst)

    ws = [W0, W1, W2, W3, W4]
    bs = [b0, b1, b2, b3, b4]
    h = xp
    for i in range(5):
        g = _tc_linear(h, ws[i], bs[i], silu_in=(i > 0))
        gp = lax.bitcast_convert_type(g.reshape(NPAD, DW, 2), jnp.int32)
        a = _agg(gp, lists, counts)
        h = lax.bitcast_convert_type(a.reshape(NPAD, DW),
                                     jnp.bfloat16).reshape(NPAD, D)
    return h[:N_NODES].astype(jnp.float32)


# speedup vs baseline: 1.1479x; 1.0187x over previous
import functools

import jax
import jax.numpy as jnp
from jax import lax
from jax.experimental import pallas as pl
from jax.experimental.pallas import tpu as pltpu
from jax.experimental.pallas import tpu_sc as plsc

N_NODES = 10000
N_EDGES = 320000
D = 128

NW = 32
NLOC = 320
NPAD = NW * NLOC
TRASH = NLOC
ACCR = NLOC + 1
LANES = 16
SUBCAP = 1024
CAP = LANES * SUBCAP
CHUNK = 128
NBUF = 10
LANES2 = 32
RPT2 = D // LANES2
DW = D // 2
ECHUNK = 16000
RPT = D // LANES


def _edge_partition_kernel():
    mesh = plsc.VectorSubcoreMesh(core_axis_name="c", subcore_axis_name="s")

    @functools.partial(
        pl.kernel,
        out_type=(
            jax.ShapeDtypeStruct((NW, CAP), jnp.int32),
            jax.ShapeDtypeStruct((NW, 128), jnp.int32),
        ),
        mesh=mesh,
        compiler_params=pltpu.CompilerParams(needs_layout_passes=False),
        scratch_types=[
            pltpu.VMEM((CAP,), jnp.int32),
            pltpu.VMEM((CAP,), jnp.int32),
            pltpu.VMEM((ECHUNK,), jnp.int32),
            pltpu.VMEM((ECHUNK,), jnp.int32),
            pltpu.VMEM((128,), jnp.int32),
        ],
    )
    def part(src_hbm, dst_hbm, lists_hbm, counts_hbm, lv, lvc, srcb, dstb,
             cv):
        wid = lax.axis_index("s") * 2 + lax.axis_index("c")
        lo = wid * NLOC
        pad = jnp.full((LANES,), TRASH, jnp.int32)

        def fill(i, _):
            lv[pl.ds(i * LANES, LANES)] = pad
            return 0

        lax.fori_loop(0, SUBCAP, fill, 0)

        lanebase = lax.iota(jnp.int32, LANES) * SUBCAP

        def outer(k, cnt):
            pltpu.sync_copy(src_hbm.at[pl.ds(k * ECHUNK, ECHUNK)], srcb)
            pltpu.sync_copy(dst_hbm.at[pl.ds(k * ECHUNK, ECHUNK)], dstb)

            def inner(j, cnt):
                sv = srcb[pl.ds(j * LANES, LANES)]
                dv = dstb[pl.ds(j * LANES, LANES)]
                dl = dv - lo
                m = (dl >= 0) & (dl < NLOC)
                pk = (sv << 9) | jnp.where(m, dl, 0)
                plsc.store_scatter(lv, [lanebase + cnt], pk, mask=m)
                return cnt + jnp.where(m, 1, 0).astype(jnp.int32)

            return lax.fori_loop(0, ECHUNK // LANES, inner, cnt)

        cnt = lax.fori_loop(0, N_EDGES // ECHUNK, outer,
                            jnp.zeros((LANES,), jnp.int32))

        base = jnp.int32(0)
        for l in range(LANES):
            nv = (cnt[l] + (LANES - 1)) >> 4
            src_base = l * SUBCAP

            def cp(j, _, b=base, s=src_base):
                lvc[pl.ds(b + j * LANES, LANES)] = lv[pl.ds(s + j * LANES,
                                                            LANES)]
                return 0

            lax.fori_loop(0, nv, cp, 0)
            base = base + nv * LANES
        for j in range(CHUNK // LANES):
            lvc[pl.ds(base + j * LANES, LANES)] = pad

        cv[pl.ds(0, LANES)] = jnp.full((LANES,), 1, jnp.int32) * base
        pltpu.sync_copy(lvc, lists_hbm.at[wid])
        pltpu.sync_copy(cv, counts_hbm.at[wid])

    return part


def _max_agg_kernel():
    mesh = plsc.VectorSubcoreMesh(core_axis_name="c", subcore_axis_name="s")

    @functools.partial(
        pl.kernel,
        out_type=jax.ShapeDtypeStruct((NPAD * DW,), jnp.int32),
        mesh=mesh,
        compiler_params=pltpu.CompilerParams(needs_layout_passes=False,
                                             use_tc_tiling_on_sc=False),
        scratch_types=[
            pltpu.VMEM((CAP,), jnp.int32),
            pltpu.VMEM((128,), jnp.int32),
            pltpu.VMEM((ACCR * DW,), jnp.int32),
        ] + [
            t for _ in range(NBUF) for t in (
                pltpu.VMEM((CHUNK,), jnp.int32),
                pltpu.VMEM((CHUNK,), jnp.int32),
                pltpu.VMEM((CHUNK, DW), jnp.int32),
                pltpu.SemaphoreType.DMA,
            )
        ],
    )
    def agg(h_hbm, lists_hbm, counts_hbm, out_hbm, lv, cv, acc, *bufargs):
        wid = lax.axis_index("s") * 2 + lax.axis_index("c")
        pltpu.sync_copy(lists_hbm.at[wid], lv)
        pltpu.sync_copy(counts_hbm.at[wid], cv)
        total = cv[pl.ds(0, LANES)][0]
        nch = (total + (CHUNK - 1)) // CHUNK

        ninf = jnp.full((LANES2,), -jnp.inf, jnp.bfloat16)
        zero = jnp.zeros((LANES2,), jnp.bfloat16)
        ninf32 = plsc.bitcast(ninf, jnp.int32)

        def initrow(rr, _):
            for k in range(RPT2):
                acc[pl.ds(rr * DW + k * LANES, LANES)] = ninf32
            return 0

        lax.fori_loop(0, ACCR, initrow, 0)

        bufs = tuple(tuple(bufargs[4 * b:4 * b + 4]) for b in range(NBUF))

        def issue(b, c):
            sidx, dloc, msg, sem = bufs[b]
            base = c * CHUNK

            def unpack(j, _):
                pk = lv[pl.ds(base + j * LANES, LANES)]
                sidx[pl.ds(j * LANES, LANES)] = pk >> 9
                dloc[pl.ds(j * LANES, LANES)] = pk & (2 ** 9 - 1)
                return 0

            lax.fori_loop(0, CHUNK // LANES, unpack, 0)
            pltpu.async_copy(h_hbm.at[sidx], msg, sem)

        for b in range(NBUF):
            @pl.when(nch > b)
            def _(b=b):
                issue(b, b)

        def louter(c2, _):
            for b in range(NBUF):
                c = c2 * NBUF + b
                sidx, dloc, msg, sem = bufs[b]

                @pl.when(c < nch)
                def _(c=c, sidx=sidx, dloc=dloc, msg=msg, sem=sem, b=b):
                    pltpu.make_async_copy(h_hbm.at[sidx], msg, sem).wait()

                    def do_group(g, _):
                        dv = dloc[pl.ds(g * LANES, LANES)]
                        for lane in range(LANES):
                            off = dv[lane] * DW
                            erow = g * LANES + lane
                            for r in range(RPT2):
                                a = plsc.bitcast(
                                    acc[pl.ds(off + r * LANES, LANES)],
                                    jnp.bfloat16)
                                mv = plsc.bitcast(
                                    msg[erow, pl.ds(r * LANES, LANES)],
                                    jnp.bfloat16)
                                acc[pl.ds(off + r * LANES, LANES)] = (
                                    plsc.bitcast(jnp.maximum(a, mv),
                                                 jnp.int32))
                        return 0

                    lax.fori_loop(0, CHUNK // LANES, do_group, 0)

                    @pl.when(c + NBUF < nch)
                    def _():
                        issue(b, c + NBUF)

            return 0

        lax.fori_loop(0, (nch + NBUF - 1) // NBUF, louter, 0)

        def fixrow(rr, _):
            for k in range(RPT2):
                v = plsc.bitcast(acc[pl.ds(rr * DW + k * LANES, LANES)],
                                 jnp.bfloat16)
                acc[pl.ds(rr * DW + k * LANES, LANES)] = (
                    plsc.bitcast(jnp.where(v == ninf, zero, v), jnp.int32))
            return 0

        lax.fori_loop(0, NLOC, fixrow, 0)
        pltpu.sync_copy(acc.at[pl.ds(0, NLOC * DW)],
                        out_hbm.at[pl.ds(wid * NLOC * DW, NLOC * DW)])

    return agg


_part = _edge_partition_kernel()
_agg = _max_agg_kernel()

_ROWS = NPAD // 4


def _tc_linear(h, w, b, silu_in):
    def body(h_ref, w_ref, b_ref, o_ref):
        hv = h_ref[...].astype(jnp.float32)
        if silu_in:
            hv = hv / (1.0 + jnp.exp(-hv))
        o_ref[...] = (lax.dot_general(
            hv, w_ref[...], (((1,), (1,)), ((), ())),
            preferred_element_type=jnp.float32,
            precision=lax.Precision.HIGHEST,
        ) + b_ref[...]).astype(jnp.bfloat16)

    return pl.pallas_call(
        body,
        grid=(NPAD // _ROWS,),
        in_specs=[
            pl.BlockSpec((_ROWS, D), lambda i: (i, 0)),
            pl.BlockSpec((D, D), lambda i: (0, 0)),
            pl.BlockSpec((1, D), lambda i: (0, 0)),
        ],
        out_specs=pl.BlockSpec((_ROWS, D), lambda i: (i, 0)),
        out_shape=jax.ShapeDtypeStruct((NPAD, D), jnp.bfloat16),
    )(h, w, b.reshape(1, D))


def kernel(x, edge_index, W0, b0, W1, b1, W2, b2, W3, b3, W4, b4):
    src = edge_index[0].astype(jnp.int32)
    dst = edge_index[1].astype(jnp.int32)
    xp = jnp.zeros((NPAD, D), jnp.float32).at[:N_NODES].set(x)

    lists, counts = _part(src, dst)

    ws = [W0, W1, W2, W3, W4]
    bs = [b0, b1, b2, b3, b4]
    h = xp
    for i in range(5):
        g = _tc_linear(h, ws[i], bs[i], silu_in=(i > 0))
        gp = lax.bitcast_convert_type(g.reshape(NPAD, DW, 2), jnp.int32)
        a = _agg(gp, lists, counts)
        h = lax.bitcast_convert_type(a.reshape(NPAD, DW),
                                     jnp.bfloat16).reshape(NPAD, D)
    return h[:N_NODES].astype(jnp.float32)


# speedup vs baseline: 1.3057x; 1.1375x over previous
import functools

import jax
import jax.numpy as jnp
from jax import lax
from jax.experimental import pallas as pl
from jax.experimental.pallas import tpu as pltpu
from jax.experimental.pallas import tpu_sc as plsc

N_NODES = 10000
N_EDGES = 320000
D = 128

NW = 32
NLOC = 320
NPAD = NW * NLOC
TRASH = NLOC
ACCR = NLOC + 1
LANES = 16
SUBCAP = 1024
CAP = LANES * SUBCAP
CHUNK = 128
NBUF = 10
LANES2 = 32
RPT2 = D // LANES2
DW = D // 2
ECHUNK = 16000
RPT = D // LANES


def _edge_partition_kernel():
    mesh = plsc.VectorSubcoreMesh(core_axis_name="c", subcore_axis_name="s")

    @functools.partial(
        pl.kernel,
        out_type=(
            jax.ShapeDtypeStruct((NW, CAP), jnp.int32),
            jax.ShapeDtypeStruct((NW, 128), jnp.int32),
        ),
        mesh=mesh,
        compiler_params=pltpu.CompilerParams(needs_layout_passes=False),
        scratch_types=[
            pltpu.VMEM((CAP,), jnp.int32),
            pltpu.VMEM((CAP,), jnp.int32),
            pltpu.VMEM((ECHUNK,), jnp.int32),
            pltpu.VMEM((ECHUNK,), jnp.int32),
            pltpu.VMEM((128,), jnp.int32),
        ],
    )
    def part(src_hbm, dst_hbm, lists_hbm, counts_hbm, lv, lvc, srcb, dstb,
             cv):
        wid = lax.axis_index("s") * 2 + lax.axis_index("c")
        lo = wid * NLOC
        pad = jnp.full((LANES,), TRASH, jnp.int32)

        def fill(i, _):
            lv[pl.ds(i * LANES, LANES)] = pad
            return 0

        lax.fori_loop(0, SUBCAP, fill, 0)

        lanebase = lax.iota(jnp.int32, LANES) * SUBCAP

        def outer(k, cnt):
            pltpu.sync_copy(src_hbm.at[pl.ds(k * ECHUNK, ECHUNK)], srcb)
            pltpu.sync_copy(dst_hbm.at[pl.ds(k * ECHUNK, ECHUNK)], dstb)

            def inner(j, cnt):
                sv = srcb[pl.ds(j * LANES, LANES)]
                dv = dstb[pl.ds(j * LANES, LANES)]
                dl = dv - lo
                m = (dl >= 0) & (dl < NLOC)
                pk = (sv << 9) | jnp.where(m, dl, 0)
                plsc.store_scatter(lv, [lanebase + cnt], pk, mask=m)
                return cnt + jnp.where(m, 1, 0).astype(jnp.int32)

            return lax.fori_loop(0, ECHUNK // LANES, inner, cnt)

        cnt = lax.fori_loop(0, N_EDGES // ECHUNK, outer,
                            jnp.zeros((LANES,), jnp.int32))

        base = jnp.int32(0)
        for l in range(LANES):
            nv = (cnt[l] + (LANES - 1)) >> 4
            src_base = l * SUBCAP

            def cp(j, _, b=base, s=src_base):
                lvc[pl.ds(b + j * LANES, LANES)] = lv[pl.ds(s + j * LANES,
                                                            LANES)]
                return 0

            lax.fori_loop(0, nv, cp, 0)
            base = base + nv * LANES
        for j in range(CHUNK // LANES):
            lvc[pl.ds(base + j * LANES, LANES)] = pad

        cv[pl.ds(0, LANES)] = jnp.full((LANES,), 1, jnp.int32) * base
        pltpu.sync_copy(lvc, lists_hbm.at[wid])
        pltpu.sync_copy(cv, counts_hbm.at[wid])

    return part


def _max_agg_kernel():
    mesh = plsc.VectorSubcoreMesh(core_axis_name="c", subcore_axis_name="s")

    @functools.partial(
        pl.kernel,
        out_type=jax.ShapeDtypeStruct((NPAD * DW,), jnp.int32),
        mesh=mesh,
        compiler_params=pltpu.CompilerParams(needs_layout_passes=False,
                                             use_tc_tiling_on_sc=False),
        scratch_types=[
            pltpu.VMEM((CAP,), jnp.int32),
            pltpu.VMEM((128,), jnp.int32),
            pltpu.VMEM((ACCR * DW,), jnp.int32),
        ] + [
            t for _ in range(NBUF) for t in (
                pltpu.VMEM((CHUNK,), jnp.int32),
                pltpu.VMEM((CHUNK,), jnp.int32),
                pltpu.VMEM((CHUNK, DW), jnp.int32),
                pltpu.SemaphoreType.DMA,
            )
        ],
    )
    def agg(h_hbm, lists_hbm, counts_hbm, out_hbm, lv, cv, acc, *bufargs):
        wid = lax.axis_index("s") * 2 + lax.axis_index("c")
        pltpu.sync_copy(lists_hbm.at[wid], lv)
        pltpu.sync_copy(counts_hbm.at[wid], cv)
        total = cv[pl.ds(0, LANES)][0]
        nch = (total + (CHUNK - 1)) // CHUNK

        ninf = jnp.full((LANES2,), -jnp.inf, jnp.bfloat16)
        zero = jnp.zeros((LANES2,), jnp.bfloat16)
        ninf32 = plsc.bitcast(ninf, jnp.int32)

        def initrow(rr, _):
            for k in range(RPT2):
                acc[pl.ds(rr * DW + k * LANES, LANES)] = ninf32
            return 0

        lax.fori_loop(0, ACCR, initrow, 0)

        bufs = tuple(tuple(bufargs[4 * b:4 * b + 4]) for b in range(NBUF))

        def issue(b, c):
            sidx, dloc, msg, sem = bufs[b]
            base = c * CHUNK

            def unpack(j, _):
                pk = lv[pl.ds(base + j * LANES, LANES)]
                sidx[pl.ds(j * LANES, LANES)] = pk >> 9
                dloc[pl.ds(j * LANES, LANES)] = pk & (2 ** 9 - 1)
                return 0

            lax.fori_loop(0, CHUNK // LANES, unpack, 0)
            pltpu.async_copy(h_hbm.at[sidx], msg, sem)

        for b in range(NBUF):
            @pl.when(nch > b)
            def _(b=b):
                issue(b, b)

        def louter(c2, _):
            for b in range(NBUF):
                c = c2 * NBUF + b
                sidx, dloc, msg, sem = bufs[b]

                @pl.when(c < nch)
                def _(c=c, sidx=sidx, dloc=dloc, msg=msg, sem=sem, b=b):
                    pltpu.make_async_copy(h_hbm.at[sidx], msg, sem).wait()

                    def do_group(g, _):
                        dv = dloc[pl.ds(g * LANES, LANES)]
                        for lane in range(LANES):
                            off = dv[lane] * DW
                            erow = g * LANES + lane
                            for r in range(RPT2):
                                a = plsc.bitcast(
                                    acc[pl.ds(off + r * LANES, LANES)],
                                    jnp.bfloat16)
                                mv = plsc.bitcast(
                                    msg[erow, pl.ds(r * LANES, LANES)],
                                    jnp.bfloat16)
                                acc[pl.ds(off + r * LANES, LANES)] = (
                                    plsc.bitcast(jnp.maximum(a, mv),
                                                 jnp.int32))
                        return 0

                    lax.fori_loop(0, CHUNK // LANES, do_group, 0)

                    @pl.when(c + NBUF < nch)
                    def _():
                        issue(b, c + NBUF)

            return 0

        lax.fori_loop(0, (nch + NBUF - 1) // NBUF, louter, 0)

        def fixrow(rr, _):
            for k in range(RPT2):
                v = plsc.bitcast(acc[pl.ds(rr * DW + k * LANES, LANES)],
                                 jnp.bfloat16)
                acc[pl.ds(rr * DW + k * LANES, LANES)] = (
                    plsc.bitcast(jnp.where(v == ninf, zero, v), jnp.int32))
            return 0

        lax.fori_loop(0, NLOC, fixrow, 0)
        pltpu.sync_copy(acc.at[pl.ds(0, NLOC * DW)],
                        out_hbm.at[pl.ds(wid * NLOC * DW, NLOC * DW)])

    return agg


_part = _edge_partition_kernel()
_agg = _max_agg_kernel()

_ROWS = NPAD // 4


def _unpack_words(au):
    lo = lax.bitcast_convert_type((au & 0xFFFF).astype(jnp.uint16),
                                  jnp.bfloat16).astype(jnp.float32)
    hi = lax.bitcast_convert_type((au >> 16).astype(jnp.uint16),
                                  jnp.bfloat16).astype(jnp.float32)
    return lo, hi


def _tc_linear(h, w, b, silu_in, packed_in):
    def _silu(v):
        return v / (1.0 + jnp.exp(-v))

    def body(h_ref, w_ref, b_ref, o_ref):
        wv = w_ref[...]
        dn = (((1,), (1,)), ((), ()))
        kw = dict(preferred_element_type=jnp.float32,
                  precision=lax.Precision.HIGHEST)
        if packed_in:
            au = lax.bitcast_convert_type(h_ref[...], jnp.uint32)
            lo, hi = _unpack_words(au)
            if silu_in:
                lo = _silu(lo)
                hi = _silu(hi)
            res = (lax.dot_general(lo, wv[:, :DW], dn, **kw)
                   + lax.dot_general(hi, wv[:, DW:], dn, **kw))
        else:
            hv = h_ref[...].astype(jnp.float32)
            if silu_in:
                hv = _silu(hv)
            res = lax.dot_general(hv, wv, dn, **kw)
        res = res + b_ref[...]
        lo_w = lax.bitcast_convert_type(res[:, :DW].astype(jnp.bfloat16),
                                        jnp.uint16).astype(jnp.uint32)
        hi_w = lax.bitcast_convert_type(res[:, DW:].astype(jnp.bfloat16),
                                        jnp.uint16).astype(jnp.uint32)
        o_ref[...] = lax.bitcast_convert_type(lo_w | (hi_w << 16),
                                              jnp.int32)

    hspec = (pl.BlockSpec((_ROWS, DW), lambda i: (i, 0)) if packed_in
             else pl.BlockSpec((_ROWS, D), lambda i: (i, 0)))
    return pl.pallas_call(
        body,
        grid=(NPAD // _ROWS,),
        in_specs=[
            hspec,
            pl.BlockSpec((D, D), lambda i: (0, 0)),
            pl.BlockSpec((1, D), lambda i: (0, 0)),
        ],
        out_specs=pl.BlockSpec((_ROWS, DW), lambda i: (i, 0)),
        out_shape=jax.ShapeDtypeStruct((NPAD, DW), jnp.int32),
    )(h, w, b.reshape(1, D))


def kernel(x, edge_index, W0, b0, W1, b1, W2, b2, W3, b3, W4, b4):
    src = edge_index[0].astype(jnp.int32)
    dst = edge_index[1].astype(jnp.int32)
    xp = jnp.zeros((NPAD, D), jnp.float32).at[:N_NODES].set(x)

    lists, counts = _part(src, dst)

    ws = [W0, W1, W2, W3, W4]
    bs = [b0, b1, b2, b3, b4]
    h = xp
    a = None
    for i in range(5):
        g = _tc_linear(h, ws[i], bs[i], silu_in=(i > 0), packed_in=(i > 0))
        a = _agg(g, lists, counts).reshape(NPAD, DW)
        h = a
    au = lax.bitcast_convert_type(a, jnp.uint32)
    lo, hi = _unpack_words(au)
    out = jnp.concatenate([lo, hi], axis=1)
    return out[:N_NODES]


# speedup vs baseline: 1.3477x; 1.0322x over previous
import functools

import jax
import jax.numpy as jnp
from jax import lax
from jax.experimental import pallas as pl
from jax.experimental.pallas import tpu as pltpu
from jax.experimental.pallas import tpu_sc as plsc

N_NODES = 10000
N_EDGES = 320000
D = 128

NW = 32
NLOC = 320
NPAD = NW * NLOC
TRASH = NLOC
ACCR = NLOC + 1
LANES = 16
SUBCAP = 1024
CAP = LANES * SUBCAP
CHUNK = 128
NBUF = 10
LANES2 = 32
RPT2 = D // LANES2
DW = D // 2
ECHUNK = 16000
RPT = D // LANES


def _edge_partition_kernel():
    mesh = plsc.VectorSubcoreMesh(core_axis_name="c", subcore_axis_name="s")

    @functools.partial(
        pl.kernel,
        out_type=(
            jax.ShapeDtypeStruct((NW, CAP), jnp.int32),
            jax.ShapeDtypeStruct((NW, 128), jnp.int32),
        ),
        mesh=mesh,
        compiler_params=pltpu.CompilerParams(needs_layout_passes=False),
        scratch_types=[
            pltpu.VMEM((CAP,), jnp.int32),
            pltpu.VMEM((CAP,), jnp.int32),
            pltpu.VMEM((ECHUNK,), jnp.int32),
            pltpu.VMEM((ECHUNK,), jnp.int32),
            pltpu.SemaphoreType.DMA,
            pltpu.VMEM((ECHUNK,), jnp.int32),
            pltpu.VMEM((ECHUNK,), jnp.int32),
            pltpu.SemaphoreType.DMA,
            pltpu.VMEM((128,), jnp.int32),
        ],
    )
    def part(src_hbm, dst_hbm, lists_hbm, counts_hbm, lv, lvc,
             srcb0, dstb0, esem0, srcb1, dstb1, esem1, cv):
        wid = lax.axis_index("s") * 2 + lax.axis_index("c")
        lo = wid * NLOC
        pad = jnp.full((LANES,), TRASH, jnp.int32)

        def fill(i, _):
            lv[pl.ds(i * LANES, LANES)] = pad
            return 0

        lax.fori_loop(0, SUBCAP, fill, 0)

        lanebase = lax.iota(jnp.int32, LANES) * SUBCAP
        NBLK = N_EDGES // ECHUNK
        ebufs = ((srcb0, dstb0, esem0), (srcb1, dstb1, esem1))

        def estage(b, k):
            srcb, dstb, esem = ebufs[b]
            pltpu.async_copy(src_hbm.at[pl.ds(k * ECHUNK, ECHUNK)], srcb,
                             esem)
            pltpu.async_copy(dst_hbm.at[pl.ds(k * ECHUNK, ECHUNK)], dstb,
                             esem)

        estage(0, 0)
        estage(1, 1)

        def outer(k2, cnt):
            for b in range(2):
                k = k2 * 2 + b
                srcb, dstb, esem = ebufs[b]
                pltpu.make_async_copy(
                    src_hbm.at[pl.ds(k * ECHUNK, ECHUNK)], srcb,
                    esem).wait()
                pltpu.make_async_copy(
                    dst_hbm.at[pl.ds(k * ECHUNK, ECHUNK)], dstb,
                    esem).wait()

                def inner(j, cnt, srcb=srcb, dstb=dstb):
                    sv = srcb[pl.ds(j * LANES, LANES)]
                    dv = dstb[pl.ds(j * LANES, LANES)]
                    dl = dv - lo
                    m = (dl >= 0) & (dl < NLOC)
                    pk = (sv << 9) | jnp.where(m, dl, 0)
                    plsc.store_scatter(lv, [lanebase + cnt], pk, mask=m)
                    return cnt + jnp.where(m, 1, 0).astype(jnp.int32)

                cnt = lax.fori_loop(0, ECHUNK // LANES, inner, cnt)

                @pl.when(k + 2 < NBLK)
                def _(b=b, k=k):
                    estage(b, k + 2)

            return cnt

        cnt = lax.fori_loop(0, NBLK // 2, outer,
                            jnp.zeros((LANES,), jnp.int32))

        base = jnp.int32(0)
        for l in range(LANES):
            nv = (cnt[l] + (LANES - 1)) >> 4
            src_base = l * SUBCAP

            def cp(j, _, b=base, s=src_base):
                lvc[pl.ds(b + j * LANES, LANES)] = lv[pl.ds(s + j * LANES,
                                                            LANES)]
                return 0

            lax.fori_loop(0, nv, cp, 0)
            base = base + nv * LANES
        for j in range(CHUNK // LANES):
            lvc[pl.ds(base + j * LANES, LANES)] = pad

        cv[pl.ds(0, LANES)] = jnp.full((LANES,), 1, jnp.int32) * base
        pltpu.sync_copy(lvc, lists_hbm.at[wid])
        pltpu.sync_copy(cv, counts_hbm.at[wid])

    return part


def _max_agg_kernel():
    mesh = plsc.VectorSubcoreMesh(core_axis_name="c", subcore_axis_name="s")

    @functools.partial(
        pl.kernel,
        out_type=jax.ShapeDtypeStruct((NPAD * DW,), jnp.int32),
        mesh=mesh,
        compiler_params=pltpu.CompilerParams(needs_layout_passes=False,
                                             use_tc_tiling_on_sc=False),
        scratch_types=[
            pltpu.VMEM((CAP,), jnp.int32),
            pltpu.VMEM((128,), jnp.int32),
            pltpu.VMEM((ACCR * DW,), jnp.int32),
        ] + [
            t for _ in range(NBUF) for t in (
                pltpu.VMEM((CHUNK,), jnp.int32),
                pltpu.VMEM((CHUNK,), jnp.int32),
                pltpu.VMEM((CHUNK, DW), jnp.int32),
                pltpu.SemaphoreType.DMA,
            )
        ],
    )
    def agg(h_hbm, lists_hbm, counts_hbm, out_hbm, lv, cv, acc, *bufargs):
        wid = lax.axis_index("s") * 2 + lax.axis_index("c")
        pltpu.sync_copy(lists_hbm.at[wid], lv)
        pltpu.sync_copy(counts_hbm.at[wid], cv)
        total = cv[pl.ds(0, LANES)][0]
        nch = (total + (CHUNK - 1)) // CHUNK

        ninf = jnp.full((LANES2,), -jnp.inf, jnp.bfloat16)
        zero = jnp.zeros((LANES2,), jnp.bfloat16)
        ninf32 = plsc.bitcast(ninf, jnp.int32)

        def initrow(rr, _):
            for k in range(RPT2):
                acc[pl.ds(rr * DW + k * LANES, LANES)] = ninf32
            return 0

        lax.fori_loop(0, ACCR, initrow, 0)

        bufs = tuple(tuple(bufargs[4 * b:4 * b + 4]) for b in range(NBUF))

        def issue(b, c):
            sidx, dloc, msg, sem = bufs[b]
            base = c * CHUNK

            def unpack(j, _):
                pk = lv[pl.ds(base + j * LANES, LANES)]
                sidx[pl.ds(j * LANES, LANES)] = pk >> 9
                dloc[pl.ds(j * LANES, LANES)] = pk & (2 ** 9 - 1)
                return 0

            lax.fori_loop(0, CHUNK // LANES, unpack, 0)
            pltpu.async_copy(h_hbm.at[sidx], msg, sem)

        for b in range(NBUF):
            @pl.when(nch > b)
            def _(b=b):
                issue(b, b)

        def louter(c2, _):
            for b in range(NBUF):
                c = c2 * NBUF + b
                sidx, dloc, msg, sem = bufs[b]

                @pl.when(c < nch)
                def _(c=c, sidx=sidx, dloc=dloc, msg=msg, sem=sem, b=b):
                    pltpu.make_async_copy(h_hbm.at[sidx], msg, sem).wait()

                    def do_group(g, _):
                        dv = dloc[pl.ds(g * LANES, LANES)]
                        for lane in range(LANES):
                            off = dv[lane] * DW
                            erow = g * LANES + lane
                            for r in range(RPT2):
                                a = plsc.bitcast(
                                    acc[pl.ds(off + r * LANES, LANES)],
                                    jnp.bfloat16)
                                mv = plsc.bitcast(
                                    msg[erow, pl.ds(r * LANES, LANES)],
                                    jnp.bfloat16)
                                acc[pl.ds(off + r * LANES, LANES)] = (
                                    plsc.bitcast(jnp.maximum(a, mv),
                                                 jnp.int32))
                        return 0

                    lax.fori_loop(0, CHUNK // LANES, do_group, 0)

                    @pl.when(c + NBUF < nch)
                    def _():
                        issue(b, c + NBUF)

            return 0

        lax.fori_loop(0, (nch + NBUF - 1) // NBUF, louter, 0)

        def fixrow(rr, _):
            for k in range(RPT2):
                v = plsc.bitcast(acc[pl.ds(rr * DW + k * LANES, LANES)],
                                 jnp.bfloat16)
                acc[pl.ds(rr * DW + k * LANES, LANES)] = (
                    plsc.bitcast(jnp.where(v == ninf, zero, v), jnp.int32))
            return 0

        lax.fori_loop(0, NLOC, fixrow, 0)
        pltpu.sync_copy(acc.at[pl.ds(0, NLOC * DW)],
                        out_hbm.at[pl.ds(wid * NLOC * DW, NLOC * DW)])

    return agg


_part = _edge_partition_kernel()
_agg = _max_agg_kernel()

_ROWS = NPAD // 4


def _unpack_words(au):
    lo = lax.bitcast_convert_type((au & 0xFFFF).astype(jnp.uint16),
                                  jnp.bfloat16).astype(jnp.float32)
    hi = lax.bitcast_convert_type((au >> 16).astype(jnp.uint16),
                                  jnp.bfloat16).astype(jnp.float32)
    return lo, hi


def _tc_linear(h, w, b, silu_in, packed_in):
    def _silu(v):
        return v / (1.0 + jnp.exp(-v))

    def body(h_ref, w_ref, b_ref, o_ref):
        wv = w_ref[...]
        dn = (((1,), (1,)), ((), ()))
        kw = dict(preferred_element_type=jnp.float32,
                  precision=lax.Precision.HIGHEST)
        if packed_in:
            au = lax.bitcast_convert_type(h_ref[...], jnp.uint32)
            lo, hi = _unpack_words(au)
            if silu_in:
                lo = _silu(lo)
                hi = _silu(hi)
            res = (lax.dot_general(lo, wv[:, :DW], dn, **kw)
                   + lax.dot_general(hi, wv[:, DW:], dn, **kw))
        else:
            hv = h_ref[...].astype(jnp.float32)
            if silu_in:
                hv = _silu(hv)
            res = lax.dot_general(hv, wv, dn, **kw)
        res = res + b_ref[...]
        lo_w = lax.bitcast_convert_type(res[:, :DW].astype(jnp.bfloat16),
                                        jnp.uint16).astype(jnp.uint32)
        hi_w = lax.bitcast_convert_type(res[:, DW:].astype(jnp.bfloat16),
                                        jnp.uint16).astype(jnp.uint32)
        o_ref[...] = lax.bitcast_convert_type(lo_w | (hi_w << 16),
                                              jnp.int32)

    hspec = (pl.BlockSpec((_ROWS, DW), lambda i: (i, 0)) if packed_in
             else pl.BlockSpec((_ROWS, D), lambda i: (i, 0)))
    return pl.pallas_call(
        body,
        grid=(NPAD // _ROWS,),
        in_specs=[
            hspec,
            pl.BlockSpec((D, D), lambda i: (0, 0)),
            pl.BlockSpec((1, D), lambda i: (0, 0)),
        ],
        out_specs=pl.BlockSpec((_ROWS, DW), lambda i: (i, 0)),
        out_shape=jax.ShapeDtypeStruct((NPAD, DW), jnp.int32),
    )(h, w, b.reshape(1, D))


def kernel(x, edge_index, W0, b0, W1, b1, W2, b2, W3, b3, W4, b4):
    src = edge_index[0].astype(jnp.int32)
    dst = edge_index[1].astype(jnp.int32)
    xp = jnp.zeros((NPAD, D), jnp.float32).at[:N_NODES].set(x)

    lists, counts = _part(src, dst)

    ws = [W0, W1, W2, W3, W4]
    bs = [b0, b1, b2, b3, b4]
    h = xp
    a = None
    for i in range(5):
        g = _tc_linear(h, ws[i], bs[i], silu_in=(i > 0), packed_in=(i > 0))
        a = _agg(g, lists, counts).reshape(NPAD, DW)
        h = a
    au = lax.bitcast_convert_type(a, jnp.uint32)
    lo, hi = _unpack_words(au)
    out = jnp.concatenate([lo, hi], axis=1)
    return out[:N_NODES]
